# baseline (speedup 1.0000x reference)
import functools

import jax
import jax.numpy as jnp
from jax import lax
from jax.experimental import pallas as pl
from jax.experimental.pallas import tpu as pltpu
from jax.experimental.pallas import tpu_sc as plsc

_NC = 2
_NS = 16
_NW = _NC * _NS
_CHUNK = 128
_PAD = 128


def kernel(user_inputs, item_inputs, user_table, item_table):
    batch = user_inputs.shape[0]
    embed_dim = user_table.shape[1]
    rows_per_view = _PAD // embed_dim
    b_per_w = batch // _NW
    n_chunks = b_per_w // _CHUNK

    utv = user_table.reshape(user_table.shape[0] // rows_per_view, _PAD)
    itv = item_table.reshape(item_table.shape[0] // rows_per_view, _PAD)

    mesh = plsc.VectorSubcoreMesh(core_axis_name="c", subcore_axis_name="s")
    out_type = (
        jax.ShapeDtypeStruct((batch, embed_dim), user_table.dtype),
        jax.ShapeDtypeStruct((batch, embed_dim), item_table.dtype),
    )

    @functools.partial(
        pl.kernel,
        mesh=mesh,
        out_type=out_type,
        scratch_types=[
            pltpu.VMEM((b_per_w,), jnp.int32),
            pltpu.VMEM((b_per_w,), jnp.int32),
            pltpu.VMEM((b_per_w,), jnp.int32),
            pltpu.VMEM((b_per_w,), jnp.int32),
            pltpu.VMEM((_CHUNK, _PAD), jnp.float32),
            pltpu.VMEM((_CHUNK, _PAD), jnp.float32),
            pltpu.VMEM((_CHUNK, embed_dim), jnp.float32),
            pltpu.VMEM((_CHUNK, embed_dim), jnp.float32),
            pltpu.SemaphoreType.DMA,
            pltpu.SemaphoreType.DMA,
        ],
    )
    def _dual_gather(ut_hbm, it_hbm, ui_hbm, ii_hbm, uo_hbm, io_hbm,
                     uidx_v, iidx_v, uq_v, iq_v, ubuf_v, ibuf_v,
                     uout_v, iout_v, usem, isem):
        wid = lax.axis_index("s") * _NC + lax.axis_index("c")
        base = wid * b_per_w

        pltpu.sync_copy(ui_hbm.at[pl.ds(base, b_per_w)], uidx_v)
        pltpu.sync_copy(ii_hbm.at[pl.ds(base, b_per_w)], iidx_v)

        @pl.loop(0, b_per_w, step=16)
        def _(k):
            uq_v[pl.ds(k, 16)] = uidx_v[pl.ds(k, 16)] >> 2
            iq_v[pl.ds(k, 16)] = iidx_v[pl.ds(k, 16)] >> 2

        for c in range(n_chunks):
            off = c * _CHUNK
            ucopy = pltpu.async_copy(
                ut_hbm.at[uq_v.at[pl.ds(off, _CHUNK)]], ubuf_v, usem
            )
            icopy = pltpu.async_copy(
                it_hbm.at[iq_v.at[pl.ds(off, _CHUNK)]], ibuf_v, isem
            )
            ucopy.wait()
            icopy.wait()

            @pl.loop(0, _CHUNK, step=16)
            def _(j0):
                uvec = (uidx_v[pl.ds(off + j0, 16)] & 3) * embed_dim
                ivec = (iidx_v[pl.ds(off + j0, 16)] & 3) * embed_dim
                for l in range(16):
                    us = uvec[l]
                    uout_v[j0 + l, pl.ds(0, 16)] = ubuf_v[
                        j0 + l, pl.ds(us, 16)
                    ]
                    uout_v[j0 + l, pl.ds(16, 16)] = ubuf_v[
                        j0 + l, pl.ds(us + 16, 16)
                    ]
                    is_ = ivec[l]
                    iout_v[j0 + l, pl.ds(0, 16)] = ibuf_v[
                        j0 + l, pl.ds(is_, 16)
                    ]
                    iout_v[j0 + l, pl.ds(16, 16)] = ibuf_v[
                        j0 + l, pl.ds(is_ + 16, 16)
                    ]

            pltpu.sync_copy(uout_v, uo_hbm.at[pl.ds(base + off, _CHUNK)])
            pltpu.sync_copy(iout_v, io_hbm.at[pl.ds(base + off, _CHUNK)])

    return _dual_gather(utv, itv, user_inputs, item_inputs)


# speedup vs baseline: 1.5144x; 1.5144x over previous
import functools

import jax
import jax.numpy as jnp
from jax import lax
from jax.experimental import pallas as pl
from jax.experimental.pallas import tpu as pltpu
from jax.experimental.pallas import tpu_sc as plsc

_NC = 2
_NS = 16
_NW = _NC * _NS
_CHUNK = 256


def kernel(user_inputs, item_inputs, user_table, item_table):
    batch = user_inputs.shape[0]
    embed_dim = user_table.shape[1]
    b_per_w = batch // _NW
    n_chunks = b_per_w // _CHUNK

    mesh = plsc.VectorSubcoreMesh(core_axis_name="c", subcore_axis_name="s")
    out_type = (
        jax.ShapeDtypeStruct((batch, embed_dim), user_table.dtype),
        jax.ShapeDtypeStruct((batch, embed_dim), item_table.dtype),
    )

    @functools.partial(
        pl.kernel,
        mesh=mesh,
        out_type=out_type,
        scratch_types=[
            pltpu.VMEM((b_per_w,), jnp.int32),
            pltpu.VMEM((b_per_w,), jnp.int32),
            pltpu.VMEM((_CHUNK, embed_dim), jnp.float32),
            pltpu.VMEM((_CHUNK, embed_dim), jnp.float32),
            pltpu.SemaphoreType.DMA,
            pltpu.SemaphoreType.DMA,
            pltpu.SemaphoreType.DMA,
            pltpu.SemaphoreType.DMA,
        ],
    )
    def _dual_gather(ut_hbm, it_hbm, ui_hbm, ii_hbm, uo_hbm, io_hbm,
                     uidx_v, iidx_v, urows_v, irows_v,
                     usem0, usem1, isem0, isem1):
        wid = lax.axis_index("s") * _NC + lax.axis_index("c")
        base = wid * b_per_w

        pltpu.sync_copy(ui_hbm.at[pl.ds(base, b_per_w)], uidx_v)
        pltpu.sync_copy(ii_hbm.at[pl.ds(base, b_per_w)], iidx_v)
        for c in range(n_chunks):
            off = c * _CHUNK

            @pl.loop(0, _CHUNK, step=16)
            def _(j0):
                uvec = uidx_v[pl.ds(off + j0, 16)]
                ivec = iidx_v[pl.ds(off + j0, 16)]
                for l in range(16):
                    pltpu.make_async_copy(
                        ut_hbm.at[pl.ds(uvec[l], 1)],
                        urows_v.at[pl.ds(j0 + l, 1)],
                        usem0 if l % 2 == 0 else usem1,
                    ).start()
                    pltpu.make_async_copy(
                        it_hbm.at[pl.ds(ivec[l], 1)],
                        irows_v.at[pl.ds(j0 + l, 1)],
                        isem0 if l % 2 == 0 else isem1,
                    ).start()

            half = _CHUNK // 2
            pltpu.make_async_copy(
                ut_hbm.at[pl.ds(0, half)], urows_v.at[pl.ds(0, half)], usem0
            ).wait()
            pltpu.make_async_copy(
                ut_hbm.at[pl.ds(0, half)], urows_v.at[pl.ds(0, half)], usem1
            ).wait()
            pltpu.sync_copy(urows_v, uo_hbm.at[pl.ds(base + off, _CHUNK)])
            pltpu.make_async_copy(
                it_hbm.at[pl.ds(0, half)], irows_v.at[pl.ds(0, half)], isem0
            ).wait()
            pltpu.make_async_copy(
                it_hbm.at[pl.ds(0, half)], irows_v.at[pl.ds(0, half)], isem1
            ).wait()
            pltpu.sync_copy(irows_v, io_hbm.at[pl.ds(base + off, _CHUNK)])

    return _dual_gather(user_table, item_table, user_inputs, item_inputs)
